# baseline (speedup 1.0000x reference)
import functools

import jax
import jax.numpy as jnp
from jax import lax
from jax.experimental import pallas as pl
from jax.experimental.pallas import tpu as pltpu
from jax.experimental.pallas import tpu_sc as plsc

E = 64
H = 768
S = 2048
A = 2 * S
T = 128
NT = A // T
G = NT + E

NC = 2
NS = 16
NW = NC * NS


_RB = 8


def _router_body(x_ref, w_ref, b_ref, pos_ref, off_ref, acol_ref):
    i = pl.program_id(0)
    rb = S // _RB

    @pl.when(i < _RB)
    def _():
        x = x_ref[...]
        w = w_ref[...]
        logits = lax.dot_general(x, w, (((1,), (1,)), ((), ())),
                                 preferred_element_type=jnp.float32)
        logits = logits + b_ref[...]
        ii = lax.broadcasted_iota(jnp.int32, logits.shape, 1)
        m1 = jnp.max(logits, axis=1, keepdims=True)
        a1 = jnp.min(jnp.where(logits == m1, ii, E), axis=1, keepdims=True)
        l2 = jnp.where(ii == a1, -jnp.inf, logits)
        m2 = jnp.max(l2, axis=1, keepdims=True)
        a2 = jnp.min(jnp.where(l2 == m2, ii, E), axis=1, keepdims=True)
        acol_ref[pl.ds(i * rb, rb), :] = a1
        acol_ref[pl.ds(S + i * rb, rb), :] = a2

    @pl.when(i == _RB)
    def _():
        ohs = []
        counts = jnp.zeros((1, E), jnp.float32)
        for j in range(A // T):
            ac = acol_ref[pl.ds(j * T, T), :]
            oh = (ac == lax.broadcasted_iota(jnp.int32, (T, E), 1)
                  ).astype(jnp.float32)
            ohs.append(oh)
            counts = counts + jnp.sum(oh, axis=0, keepdims=True)
        iu = lax.broadcasted_iota(jnp.int32, (E, E), 0)
        ustrict = (iu < lax.broadcasted_iota(jnp.int32, (E, E), 1)
                   ).astype(jnp.float32)
        off_row = lax.dot_general(counts, ustrict, (((1,), (0,)), ((), ())),
                                  precision=lax.Precision.HIGHEST,
                                  preferred_element_type=jnp.float32)

        il = lax.broadcasted_iota(jnp.int32, (T, T), 0)
        lstrict = (il > lax.broadcasted_iota(jnp.int32, (T, T), 1)
                   ).astype(jnp.float32)
        eye = (il == lax.broadcasted_iota(jnp.int32, (T, T), 1)
               ).astype(jnp.float32)

        run = jnp.zeros((1, E), jnp.float32)
        for j in range(A // T):
            oh = ohs[j]
            pre = lax.dot_general(lstrict, oh, (((1,), (0,)), ((), ())),
                                  precision=lax.Precision.HIGHEST,
                                  preferred_element_type=jnp.float32) + run
            rank = jnp.sum(pre * oh, axis=1, keepdims=True)
            start = jnp.sum(off_row * oh, axis=1, keepdims=True)
            pos_col = rank + start
            pos_row = lax.dot_general(pos_col, eye, (((0,), (0,)), ((), ())),
                                      precision=lax.Precision.HIGHEST,
                                      preferred_element_type=jnp.float32)
            pos_ref[pl.ds(j, 1), :] = pos_row.astype(jnp.int32)
            run = run + jnp.sum(oh, axis=0, keepdims=True)
        off_pad = jnp.concatenate(
            [off_row, jnp.full((1, E), float(A), jnp.float32)], axis=1)
        off_ref[...] = off_pad.astype(jnp.int32)


def _route(x2d, router_W, router_b):
    rb = S // _RB
    return pl.pallas_call(
        _router_body,
        grid=(_RB + 1,),
        in_specs=[
            pl.BlockSpec((rb, H), lambda i: (jnp.minimum(i, _RB - 1), 0)),
            pl.BlockSpec((E, H), lambda i: (0, 0)),
            pl.BlockSpec((1, E), lambda i: (0, 0)),
        ],
        out_specs=[
            pl.BlockSpec((A // T, T), lambda i: (0, 0)),
            pl.BlockSpec((1, 2 * E), lambda i: (0, 0)),
        ],
        out_shape=[
            jax.ShapeDtypeStruct((A // T, T), jnp.int32),
            jax.ShapeDtypeStruct((1, 2 * E), jnp.int32),
        ],
        scratch_shapes=[pltpu.VMEM((A, 1), jnp.int32)],
    )(x2d, router_W, router_b.reshape(1, E))


def _scatter_body(x_hbm, idx_hbm, out_hbm, idx_v, rows_v, sems, semw):
    wid = lax.axis_index("s") * NC + lax.axis_index("c")
    rows = A // NW
    crows = rows // _KC
    base = wid * rows
    pltpu.sync_copy(idx_hbm.at[pl.ds(base, rows)], idx_v)
    gets = []
    for c in range(_KC):
        sl = pl.ds(c * crows, crows)
        gets.append(pltpu.async_copy(x_hbm.at[idx_v.at[sl]],
                                     rows_v.at[sl], sems[c]))
    puts = []
    for c in range(_KC):
        gets[c].wait()
        sl = pl.ds(c * crows, crows)
        puts.append(pltpu.async_copy(
            rows_v.at[sl], out_hbm.at[pl.ds(base + c * crows, crows)],
            semw[c]))
    for p in puts:
        p.wait()


def _scatter_sorted(x2d, sorted_tok):
    mesh = plsc.VectorSubcoreMesh(core_axis_name="c", subcore_axis_name="s",
                                  num_cores=NC, num_subcores=NS)
    rows = A // NW
    return pl.kernel(
        _scatter_body,
        out_type=jax.ShapeDtypeStruct((A, H), jnp.float32),
        mesh=mesh,
        scratch_types=[
            pltpu.VMEM((rows,), jnp.int32),
            pltpu.VMEM((rows, H), jnp.float32),
            [pltpu.SemaphoreType.DMA for _ in range(_KC)],
            [pltpu.SemaphoreType.DMA for _ in range(_KC)],
        ],
    )(x2d, sorted_tok)


def _gmm_body(off_ref, xs_ref, w_ref, b_ref, out_ref, acc_ref, sem):
    e = pl.program_id(0)
    start = off_ref[e]
    end = off_ref[e + 1]
    w0 = start // T
    nwin = (end + T - 1) // T - w0
    wm = w_ref[0]
    bias = b_ref[0]

    def body(k, _):
        r0 = (w0 + k) * T
        x = xs_ref[pl.ds(r0, T), :]
        y = lax.dot_general(x, wm, (((1,), (1,)), ((), ())),
                            preferred_element_type=jnp.float32) + bias
        r = lax.broadcasted_iota(jnp.int32, (T, 1), 0) + r0
        mask = (r >= start) & (r < end)
        acc_ref[pl.ds(r0, T), :] = jnp.where(mask, y, acc_ref[pl.ds(r0, T), :])

        @pl.when(end >= r0 + T)
        def _():
            pltpu.async_copy(acc_ref.at[pl.ds(r0, T), :],
                             out_ref.at[pl.ds(r0, T), :], sem)

        return 0

    lax.fori_loop(0, nwin, body, 0)

    @pl.when(e == E - 1)
    def _():
        for w in range(NT):
            pltpu.make_async_copy(acc_ref.at[pl.ds(w * T, T), :],
                                  out_ref.at[pl.ds(w * T, T), :], sem).wait()


def _grouped_matmul(xs, expert_W, expert_b, off):
    grid_spec = pltpu.PrefetchScalarGridSpec(
        num_scalar_prefetch=1,
        grid=(E,),
        in_specs=[
            pl.BlockSpec((A, H), lambda e, off: (0, 0)),
            pl.BlockSpec((1, H, H), lambda e, off: (e, 0, 0)),
            pl.BlockSpec((1, 1, H), lambda e, off: (e, 0, 0)),
        ],
        out_specs=pl.BlockSpec(memory_space=pltpu.MemorySpace.HBM),
        scratch_shapes=[
            pltpu.VMEM((A, H), jnp.float32),
            pltpu.SemaphoreType.DMA,
        ],
    )
    return pl.pallas_call(
        _gmm_body,
        grid_spec=grid_spec,
        out_shape=jax.ShapeDtypeStruct((A, H), jnp.float32),
    )(off, xs, expert_W, expert_b.reshape(E, 1, H))


_KC = 4


def _combine_body(y_hbm, pos_hbm, out_hbm,
                  i1_v, i2_v, y1_v, y2_v, sems, semw):
    wid = lax.axis_index("s") * NC + lax.axis_index("c")
    rows = S // NW
    crows = rows // _KC
    base = wid * rows
    pltpu.sync_copy(pos_hbm.at[pl.ds(base, rows)], i1_v)
    pltpu.sync_copy(pos_hbm.at[pl.ds(S + base, rows)], i2_v)

    gets = []
    for c in range(_KC):
        sl = pl.ds(c * crows, crows)
        gets.append((
            pltpu.async_copy(y_hbm.at[i1_v.at[sl]], y1_v.at[sl], sems[c][0]),
            pltpu.async_copy(y_hbm.at[i2_v.at[sl]], y2_v.at[sl], sems[c][1]),
        ))

    puts = []
    for c in range(_KC):
        gets[c][0].wait()
        gets[c][1].wait()

        @plsc.parallel_loop(c * crows, (c + 1) * crows, unroll=4)
        def _(r):
            for cc in range(0, H, 16):
                y1_v[r, pl.ds(cc, 16)] = (y1_v[r, pl.ds(cc, 16)]
                                          + y2_v[r, pl.ds(cc, 16)])
        sl = pl.ds(c * crows, crows)
        puts.append(pltpu.async_copy(
            y1_v.at[sl], out_hbm.at[pl.ds(base + c * crows, crows)], semw[c]))
    for p in puts:
        p.wait()


def _combine(ys, pos):
    mesh = plsc.VectorSubcoreMesh(core_axis_name="c", subcore_axis_name="s",
                                  num_cores=NC, num_subcores=NS)
    rows = S // NW
    return pl.kernel(
        _combine_body,
        out_type=jax.ShapeDtypeStruct((S, H), jnp.float32),
        mesh=mesh,
        scratch_types=[
            pltpu.VMEM((rows,), jnp.int32),
            pltpu.VMEM((rows,), jnp.int32),
            pltpu.VMEM((rows, H), jnp.float32),
            pltpu.VMEM((rows, H), jnp.float32),
            [[pltpu.SemaphoreType.DMA, pltpu.SemaphoreType.DMA]
             for _ in range(_KC)],
            [pltpu.SemaphoreType.DMA for _ in range(_KC)],
        ],
    )(ys, pos)


@jax.jit
def kernel(hidden_states, router_W, router_b, expert_W, expert_b):
    x2d = hidden_states.reshape(S, H)

    posr, offr = _route(x2d, router_W, router_b)
    pos = posr.reshape(A)
    off = offr.reshape(2 * E)

    tok = jnp.arange(S, dtype=jnp.int32)
    sorted_tok = jnp.zeros((A,), jnp.int32).at[pos].set(
        jnp.concatenate([tok, tok]))
    xs = _scatter_sorted(x2d, sorted_tok)
    ys = _grouped_matmul(xs, expert_W, expert_b, off)
    out2d = _combine(ys, pos)
    return out2d.reshape(1, S, H)


# speedup vs baseline: 1.0209x; 1.0209x over previous
import functools

import jax
import jax.numpy as jnp
from jax import lax
from jax.experimental import pallas as pl
from jax.experimental.pallas import tpu as pltpu
from jax.experimental.pallas import tpu_sc as plsc

E = 64
H = 768
S = 2048
A = 2 * S
T = 128
NT = A // T
G = NT + E

NC = 2
NS = 16
NW = NC * NS


_RB = 8


def _router_body(x_ref, w_ref, b_ref, pos_ref, off_ref, acol_ref):
    i = pl.program_id(0)
    rb = S // _RB

    @pl.when(i < _RB)
    def _():
        x = x_ref[...]
        w = w_ref[...]
        logits = lax.dot_general(x, w, (((1,), (1,)), ((), ())),
                                 preferred_element_type=jnp.float32)
        logits = logits + b_ref[...]
        ii = lax.broadcasted_iota(jnp.int32, logits.shape, 1)
        m1 = jnp.max(logits, axis=1, keepdims=True)
        a1 = jnp.min(jnp.where(logits == m1, ii, E), axis=1, keepdims=True)
        l2 = jnp.where(ii == a1, -jnp.inf, logits)
        m2 = jnp.max(l2, axis=1, keepdims=True)
        a2 = jnp.min(jnp.where(l2 == m2, ii, E), axis=1, keepdims=True)
        acol_ref[pl.ds(i * rb, rb), :] = a1
        acol_ref[pl.ds(S + i * rb, rb), :] = a2

    @pl.when(i == _RB)
    def _():
        ohs = []
        counts = jnp.zeros((1, E), jnp.float32)
        for j in range(A // T):
            ac = acol_ref[pl.ds(j * T, T), :]
            oh = (ac == lax.broadcasted_iota(jnp.int32, (T, E), 1)
                  ).astype(jnp.float32)
            ohs.append(oh)
            counts = counts + jnp.sum(oh, axis=0, keepdims=True)
        iu = lax.broadcasted_iota(jnp.int32, (E, E), 0)
        ustrict = (iu < lax.broadcasted_iota(jnp.int32, (E, E), 1)
                   ).astype(jnp.float32)
        off_row = lax.dot_general(counts, ustrict, (((1,), (0,)), ((), ())),
                                  precision=lax.Precision.HIGHEST,
                                  preferred_element_type=jnp.float32)

        il = lax.broadcasted_iota(jnp.int32, (T, T), 0)
        lstrict = (il > lax.broadcasted_iota(jnp.int32, (T, T), 1)
                   ).astype(jnp.float32)
        eye = (il == lax.broadcasted_iota(jnp.int32, (T, T), 1)
               ).astype(jnp.float32)

        run = jnp.zeros((1, E), jnp.float32)
        for j in range(A // T):
            oh = ohs[j]
            pre = lax.dot_general(lstrict, oh, (((1,), (0,)), ((), ())),
                                  precision=lax.Precision.HIGHEST,
                                  preferred_element_type=jnp.float32) + run
            rank = jnp.sum(pre * oh, axis=1, keepdims=True)
            start = jnp.sum(off_row * oh, axis=1, keepdims=True)
            pos_col = rank + start
            pos_row = lax.dot_general(pos_col, eye, (((0,), (0,)), ((), ())),
                                      precision=lax.Precision.HIGHEST,
                                      preferred_element_type=jnp.float32)
            pos_ref[pl.ds(j, 1), :] = pos_row.astype(jnp.int32)
            run = run + jnp.sum(oh, axis=0, keepdims=True)
        off_pad = jnp.concatenate(
            [off_row, jnp.full((1, E), float(A), jnp.float32)], axis=1)
        off_ref[...] = off_pad.astype(jnp.int32)


def _route(x2d, router_W, router_b):
    rb = S // _RB
    return pl.pallas_call(
        _router_body,
        grid=(_RB + 1,),
        in_specs=[
            pl.BlockSpec((rb, H), lambda i: (jnp.minimum(i, _RB - 1), 0)),
            pl.BlockSpec((E, H), lambda i: (0, 0)),
            pl.BlockSpec((1, E), lambda i: (0, 0)),
        ],
        out_specs=[
            pl.BlockSpec((A // T, T), lambda i: (0, 0)),
            pl.BlockSpec((1, 2 * E), lambda i: (0, 0)),
        ],
        out_shape=[
            jax.ShapeDtypeStruct((A // T, T), jnp.int32),
            jax.ShapeDtypeStruct((1, 2 * E), jnp.int32),
        ],
        scratch_shapes=[pltpu.VMEM((A, 1), jnp.int32)],
    )(x2d, router_W, router_b.reshape(1, E))


def _scatter_body(x_hbm, idx_hbm, out_hbm, idx_v, rows_v, sems, semw):
    wid = lax.axis_index("s") * NC + lax.axis_index("c")
    rows = A // NW
    crows = rows // _KC
    base = wid * rows
    pltpu.sync_copy(idx_hbm.at[pl.ds(base, rows)], idx_v)
    gets = []
    for c in range(_KC):
        sl = pl.ds(c * crows, crows)
        gets.append(pltpu.async_copy(x_hbm.at[idx_v.at[sl]],
                                     rows_v.at[sl], sems[c]))
    puts = []
    for c in range(_KC):
        gets[c].wait()
        sl = pl.ds(c * crows, crows)
        puts.append(pltpu.async_copy(
            rows_v.at[sl], out_hbm.at[pl.ds(base + c * crows, crows)],
            semw[c]))
    for p in puts:
        p.wait()


def _scatter_sorted(x2d, sorted_tok):
    mesh = plsc.VectorSubcoreMesh(core_axis_name="c", subcore_axis_name="s",
                                  num_cores=NC, num_subcores=NS)
    rows = A // NW
    return pl.kernel(
        _scatter_body,
        out_type=jax.ShapeDtypeStruct((A, H), jnp.float32),
        mesh=mesh,
        scratch_types=[
            pltpu.VMEM((rows,), jnp.int32),
            pltpu.VMEM((rows, H), jnp.float32),
            [pltpu.SemaphoreType.DMA for _ in range(_KC)],
            [pltpu.SemaphoreType.DMA for _ in range(_KC)],
        ],
    )(x2d, sorted_tok)


def _gmm_body(off_ref, xs_ref, w_ref, b_ref, out_ref, acc_ref, sem):
    e = pl.program_id(0)
    start = off_ref[e]
    end = off_ref[e + 1]
    w0 = start // T
    nwin = (end + T - 1) // T - w0
    wm = w_ref[0]
    bias = b_ref[0]

    def body(k, _):
        r0 = (w0 + k) * T
        x = xs_ref[pl.ds(r0, T), :]
        y = lax.dot_general(x, wm, (((1,), (1,)), ((), ())),
                            preferred_element_type=jnp.float32) + bias
        r = lax.broadcasted_iota(jnp.int32, (T, 1), 0) + r0
        mask = (r >= start) & (r < end)
        acc_ref[pl.ds(r0, T), :] = jnp.where(mask, y, acc_ref[pl.ds(r0, T), :])

        @pl.when(end >= r0 + T)
        def _():
            pltpu.async_copy(acc_ref.at[pl.ds(r0, T), :],
                             out_ref.at[pl.ds(r0, T), :], sem)

        return 0

    lax.fori_loop(0, nwin, body, 0)

    @pl.when(e == E - 1)
    def _():
        for w in range(NT):
            pltpu.make_async_copy(acc_ref.at[pl.ds(w * T, T), :],
                                  out_ref.at[pl.ds(w * T, T), :], sem).wait()


def _grouped_matmul(xs, expert_W, expert_b, off):
    grid_spec = pltpu.PrefetchScalarGridSpec(
        num_scalar_prefetch=1,
        grid=(E,),
        in_specs=[
            pl.BlockSpec((A, H), lambda e, off: (0, 0)),
            pl.BlockSpec((1, H, H), lambda e, off: (e, 0, 0)),
            pl.BlockSpec((1, 1, H), lambda e, off: (e, 0, 0)),
        ],
        out_specs=pl.BlockSpec(memory_space=pltpu.MemorySpace.HBM),
        scratch_shapes=[
            pltpu.VMEM((A, H), jnp.float32),
            pltpu.SemaphoreType.DMA,
        ],
    )
    return pl.pallas_call(
        _gmm_body,
        grid_spec=grid_spec,
        out_shape=jax.ShapeDtypeStruct((A, H), jnp.float32),
    )(off, xs, expert_W, expert_b.reshape(E, 1, H))


_KC = 4


def _combine_body(y_hbm, pos_hbm, out_hbm,
                  i1_v, i2_v, y1_v, y2_v, sems, semw):
    wid = lax.axis_index("s") * NC + lax.axis_index("c")
    rows = S // NW
    crows = rows // _KC
    base = wid * rows
    pltpu.sync_copy(pos_hbm.at[pl.ds(base, rows)], i1_v)
    pltpu.sync_copy(pos_hbm.at[pl.ds(S + base, rows)], i2_v)

    gets = []
    for c in range(_KC):
        sl = pl.ds(c * crows, crows)
        gets.append((
            pltpu.async_copy(y_hbm.at[i1_v.at[sl]], y1_v.at[sl], sems[c][0]),
            pltpu.async_copy(y_hbm.at[i2_v.at[sl]], y2_v.at[sl], sems[c][1]),
        ))

    puts = []
    for c in range(_KC):
        gets[c][0].wait()
        gets[c][1].wait()

        def addrow(r, _, c=c):
            for cc in range(0, H, 16):
                y1_v[r, pl.ds(cc, 16)] = (y1_v[r, pl.ds(cc, 16)]
                                          + y2_v[r, pl.ds(cc, 16)])
            return 0

        lax.fori_loop(c * crows, (c + 1) * crows, addrow, 0)
        sl = pl.ds(c * crows, crows)
        puts.append(pltpu.async_copy(
            y1_v.at[sl], out_hbm.at[pl.ds(base + c * crows, crows)], semw[c]))
    for p in puts:
        p.wait()


def _combine(ys, pos):
    mesh = plsc.VectorSubcoreMesh(core_axis_name="c", subcore_axis_name="s",
                                  num_cores=NC, num_subcores=NS)
    rows = S // NW
    return pl.kernel(
        _combine_body,
        out_type=jax.ShapeDtypeStruct((S, H), jnp.float32),
        mesh=mesh,
        scratch_types=[
            pltpu.VMEM((rows,), jnp.int32),
            pltpu.VMEM((rows,), jnp.int32),
            pltpu.VMEM((rows, H), jnp.float32),
            pltpu.VMEM((rows, H), jnp.float32),
            [[pltpu.SemaphoreType.DMA, pltpu.SemaphoreType.DMA]
             for _ in range(_KC)],
            [pltpu.SemaphoreType.DMA for _ in range(_KC)],
        ],
    )(ys, pos)


@jax.jit
def kernel(hidden_states, router_W, router_b, expert_W, expert_b):
    x2d = hidden_states.reshape(S, H)

    posr, offr = _route(x2d, router_W, router_b)
    pos = posr.reshape(A)
    off = offr.reshape(2 * E)

    tok = jnp.arange(S, dtype=jnp.int32)
    sorted_tok = jnp.zeros((A,), jnp.int32).at[pos].set(
        jnp.concatenate([tok, tok]))
    xs = _scatter_sorted(x2d, sorted_tok)
    ys = _grouped_matmul(xs, expert_W, expert_b, off)
    out2d = _combine(ys, pos)
    return out2d.reshape(1, S, H)
